# baseline (speedup 1.0000x reference)
import functools

import jax
import jax.numpy as jnp
from jax import lax
from jax.experimental import pallas as pl
from jax.experimental.pallas import tpu as pltpu
from jax.experimental.pallas import tpu_sc as plsc


def _sc_build(s_len, n0, n1, d0, d1, nc, ns):
    nw = nc * ns
    rows_per_w = s_len // nw
    ck = 16
    n_chunks = rows_per_w // ck

    mesh = plsc.VectorSubcoreMesh(core_axis_name="c", subcore_axis_name="s")

    @functools.partial(
        pl.kernel,
        out_type=jax.ShapeDtypeStruct((s_len, d0 + d1), jnp.float32),
        mesh=mesh,
        scratch_types=[
            pltpu.VMEM_SHARED((n0, d0), jnp.float32),
            pltpu.VMEM((ck, d0 + d1), jnp.float32),
            pltpu.VMEM((ck, d0 + d1), jnp.float32),
            pltpu.SemaphoreType.DMA,
            pltpu.SemaphoreType.DMA,
            pltpu.SemaphoreType.DMA,
            pltpu.SemaphoreType.DMA,
            pltpu.SemaphoreType.DMA,
            pltpu.SemaphoreType.DMA,
        ],
    )
    def body(x1_hbm, x2_hbm, out_hbm, x1_sh, b0, b1, sa0, sa1, sg0, sg1, sw0, sw1):
        wid = lax.axis_index("s") * nc + lax.axis_index("c")
        bufs = (b0, b1)
        asems = (sa0, sa1)
        gsems = (sg0, sg1)
        wsems = (sw0, sw1)

        @pl.when(lax.axis_index("s") == 0)
        def _():
            pltpu.sync_copy(x1_hbm, x1_sh)

        plsc.subcore_barrier()

        w0 = wid * rows_per_w
        writes = [None, None]
        for k in range(n_chunks):
            b = k % 2
            if writes[b] is not None:
                writes[b].wait()
            base = w0 + k * ck
            j = wid * (rows_per_w // n0) + k // (n0 // ck)
            jvec = jnp.full((16,), j, jnp.int32)
            buf = bufs[b]
            c1 = pltpu.async_copy(
                x1_sh.at[pl.ds((k % (n0 // ck)) * ck, ck)],
                buf.at[:, pl.ds(0, d0)],
                asems[b],
            )
            c2 = pltpu.async_copy(x2_hbm.at[jvec], buf.at[:, pl.ds(d0, d1)], gsems[b])
            c1.wait()
            c2.wait()
            writes[b] = pltpu.async_copy(buf, out_hbm.at[pl.ds(base, ck)], wsems[b])

        for w in writes:
            if w is not None:
                w.wait()

    return body


def kernel(x, x1, x2):
    s_len = x.shape[1]
    n0, d0 = x1.shape
    n1, d1 = x2.shape
    info = plsc.get_sparse_core_info()
    build = _sc_build(s_len, n0, n1, d0, d1, info.num_cores, info.num_subcores)
    out = build(x1, x2)
    return out.astype(x.dtype)[None, :, :]


# speedup vs baseline: 2.0455x; 2.0455x over previous
import functools

import jax
import jax.numpy as jnp
from jax import lax
from jax.experimental import pallas as pl
from jax.experimental.pallas import tpu as pltpu
from jax.experimental.pallas import tpu_sc as plsc


def _sc_build(s_len, n0, n1, d0, d1, nc, ns):
    nw = nc * ns
    rows_per_w = s_len // nw
    j_per_w = n1 // nw
    rep = 16

    mesh = plsc.VectorSubcoreMesh(core_axis_name="c", subcore_axis_name="s")

    @functools.partial(
        pl.kernel,
        out_type=jax.ShapeDtypeStruct((s_len, d0 + d1), jnp.float32),
        mesh=mesh,
        scratch_types=[
            pltpu.VMEM_SHARED((j_per_w * n0, d0), jnp.float32),
            pltpu.VMEM((rep, d1), jnp.float32),
            pltpu.VMEM((rep, d1), jnp.float32),
            pltpu.VMEM((rep, d1), jnp.float32),
            pltpu.VMEM((rep, d1), jnp.float32),
            pltpu.SemaphoreType.DMA,
            pltpu.SemaphoreType.DMA,
            pltpu.SemaphoreType.DMA,
        ],
    )
    def body(x1_hbm, x2_hbm, out_hbm, x1r_sh, g0, g1, g2, g3, sg, sx, sw):
        sid = lax.axis_index("s")
        wid = sid * nc + lax.axis_index("c")
        gbufs = (g0, g1, g2, g3)

        gathers = []
        for t in range(j_per_w):
            j = wid * j_per_w + t
            jvec = jnp.full((16,), j, jnp.int32)
            gathers.append(pltpu.async_copy(x2_hbm.at[jvec], gbufs[t], sg))

        @pl.when(sid < j_per_w)
        def _():
            pltpu.sync_copy(x1_hbm, x1r_sh.at[pl.ds(sid * n0, n0)])

        plsc.subcore_barrier()

        w0 = wid * rows_per_w
        xw = pltpu.async_copy(
            x1r_sh, out_hbm.at[pl.ds(w0, rows_per_w), pl.ds(0, d0)], sx
        )

        writes = []
        for t in range(j_per_w):
            gathers[t].wait()
            base = (wid * j_per_w + t) * n0
            for h in range(n0 // rep):
                writes.append(
                    pltpu.async_copy(
                        gbufs[t],
                        out_hbm.at[pl.ds(base + h * rep, rep), pl.ds(d0, d1)],
                        sw,
                    )
                )
        for w in writes:
            w.wait()
        xw.wait()

    return body


def kernel(x, x1, x2):
    s_len = x.shape[1]
    n0, d0 = x1.shape
    n1, d1 = x2.shape
    info = plsc.get_sparse_core_info()
    build = _sc_build(s_len, n0, n1, d0, d1, info.num_cores, info.num_subcores)
    out = build(x1, x2)
    return out.astype(x.dtype)[None, :, :]
